# baseline (speedup 1.0000x reference)
import functools

import jax
import jax.numpy as jnp
from jax import lax
from jax.experimental import pallas as pl
from jax.experimental.pallas import tpu as pltpu
from jax.experimental.pallas import tpu_sc as plsc

_NUM_CORES = 2
_NUM_SUBCORES = 16
_NUM_WORKERS = _NUM_CORES * _NUM_SUBCORES
_LANES = 16
_HI_MASK = -65536
_BLK = 2 * _LANES


def _rne16(x):
    return lax.shift_right_logical(
        x + 0x7FFF + (lax.shift_right_logical(x, 16) & 1), 16)


@functools.lru_cache(maxsize=None)
def _make_gather_sum(L, B, V):
    assert L % 8 == 0 and B == 1024 and V % _BLK == 0
    tile_rows = L // 8
    b_per_w = B // _NUM_WORKERS
    groups = b_per_w // _LANES
    VM = (V // 128) * 128
    VT = V - VM
    assert VT % _BLK == 0
    rows = VM // 128
    n_blocks_main = VM // _BLK
    n_tail_blk = VT // _BLK
    blk_per_tile = 200
    f32_per_tile = blk_per_tile * _BLK
    rows_per_tile = f32_per_tile // 128
    words_per_tile = blk_per_tile * _LANES
    stage_w = _NUM_SUBCORES * words_per_tile
    tail_owner = n_blocks_main // blk_per_tile
    assert (n_blocks_main - tail_owner * blk_per_tile) + n_tail_blk \
        <= blk_per_tile
    mesh = plsc.VectorSubcoreMesh(core_axis_name="c", subcore_axis_name="s")

    @functools.partial(
        pl.kernel,
        mesh=mesh,
        out_type=(
            jax.ShapeDtypeStruct((1, B), jnp.float32),
            jax.ShapeDtypeStruct((_NUM_CORES, stage_w), jnp.int32),
        ),
        scratch_types=[
            pltpu.VMEM((V // 2,), jnp.int32),
            pltpu.VMEM((tile_rows, 8, b_per_w), jnp.int32),
            pltpu.VMEM((rows_per_tile, 128), jnp.float32),
            pltpu.VMEM((words_per_tile,), jnp.int32),
            pltpu.VMEM((VT,), jnp.float32),
            pltpu.VMEM((b_per_w,), jnp.float32),
            pltpu.VMEM((_LANES,), jnp.float32),
            pltpu.SemaphoreType.DMA,
            pltpu.SemaphoreType.DMA,
        ],
        compiler_params=pltpu.CompilerParams(
            needs_layout_passes=False, use_tc_tiling_on_sc=False),
    )
    def gather_sum(w_hbm, wt_hbm, tex_hbm, bias_hbm, out_hbm, stage_hbm,
                   w_v, tex_v, pin_v, pout_v, tail_v, out_v, bias_v,
                   sem_w, sem_t):
        core = lax.axis_index("c")
        sid = lax.axis_index("s")
        wid = sid * _NUM_CORES + core
        tile_col = wid // 4
        c0 = (wid % 4) * b_per_w
        cp_t = pltpu.async_copy(
            tex_hbm.at[:, tile_col, :, pl.ds(c0, b_per_w)], tex_v, sem_t)
        pltpu.sync_copy(bias_hbm, bias_v.at[pl.ds(0, 1)])

        blk0 = sid * blk_per_tile
        row_want = sid * rows_per_tile
        row_start = jnp.minimum(row_want, rows - rows_per_tile)
        local_off = (row_want - row_start) * 128
        pltpu.sync_copy(w_hbm.at[pl.ds(row_start, rows_per_tile), :], pin_v)
        pltpu.sync_copy(wt_hbm.at[0], tail_v)

        def pack_body(j, carry):
            for u in range(4):
                i = j * 4 + u

                @pl.when(blk0 + i < n_blocks_main)
                def _(i=i):
                    base = local_off + i * _BLK
                    r = base >> 7
                    c = base & 127
                    b0 = plsc.bitcast(pin_v[r, pl.ds(c, _LANES)], jnp.int32)
                    b1 = plsc.bitcast(pin_v[r, pl.ds(c + _LANES, _LANES)],
                                      jnp.int32)
                    pout_v[pl.ds(i * _LANES, _LANES)] = (
                        _rne16(b0) | (_rne16(b1) << 16))
            return carry

        lax.fori_loop(0, blk_per_tile // 4, pack_body, 0)

        @pl.when(sid == tail_owner)
        def _():
            lo = n_blocks_main - tail_owner * blk_per_tile
            for t in range(n_tail_blk):
                b0 = plsc.bitcast(tail_v[pl.ds(t * _BLK, _LANES)], jnp.int32)
                b1 = plsc.bitcast(tail_v[pl.ds(t * _BLK + _LANES, _LANES)],
                                  jnp.int32)
                pout_v[pl.ds((lo + t) * _LANES, _LANES)] = (
                    _rne16(b0) | (_rne16(b1) << 16))

        pltpu.sync_copy(pout_v,
                        stage_hbm.at[core, pl.ds(sid * words_per_tile,
                                                 words_per_tile)])
        plsc.subcore_barrier()

        cp_w = pltpu.async_copy(stage_hbm.at[core, pl.ds(0, V // 2)], w_v,
                                sem_w)
        cp_t.wait()
        cp_w.wait()
        bias = bias_v[...][0]

        def body(tr, accs):
            out = list(accs)
            for sub in range(8):
                for g in range(groups):
                    idx = tex_v[tr, sub, pl.ds(g * _LANES, _LANES)]
                    gidx = ((idx >> 5) << 4) | (idx & 15)
                    word = plsc.load_gather(w_v, [gidx])
                    bits = jnp.where((idx & 16) == 16, word & _HI_MASK,
                                     word << 16)
                    out[g] = out[g] + plsc.bitcast(bits, jnp.float32)
            return tuple(out)

        init = tuple(jnp.zeros((_LANES,), jnp.float32) for _ in range(groups))
        accs = lax.fori_loop(0, tile_rows, body, init)
        for g in range(groups):
            out_v[pl.ds(g * _LANES, _LANES)] = accs[g] + bias
        pltpu.sync_copy(
            out_v, out_hbm.at[0, pl.ds(tile_col * 128 + c0, b_per_w)])

    return gather_sum


def kernel(text, W, b):
    L, B = text.shape
    V = W.shape[1]
    tex4 = text.reshape(L // 8, 8, B // 128, 128).transpose(0, 2, 1, 3)
    VM = (V // 128) * 128
    w_main = lax.slice(W, (0, 0), (1, VM)).reshape(VM // 128, 128)
    w_tail = lax.slice(W, (0, VM), (1, V))
    out, _ = _make_gather_sum(L, B, V)(w_main, w_tail, tex4, b)
    return out.reshape(B, 1)


# speedup vs baseline: 1.0732x; 1.0732x over previous
import functools

import jax
import jax.numpy as jnp
from jax import lax
from jax.experimental import pallas as pl
from jax.experimental.pallas import tpu as pltpu
from jax.experimental.pallas import tpu_sc as plsc

_NUM_CORES = 2
_NUM_SUBCORES = 16
_NUM_WORKERS = _NUM_CORES * _NUM_SUBCORES
_LANES = 16
_HI_MASK = -65536
_BLK = 2 * _LANES


def _rne16(x):
    return lax.shift_right_logical(
        x + 0x7FFF + (lax.shift_right_logical(x, 16) & 1), 16)


@functools.lru_cache(maxsize=None)
def _make_gather_sum(L, B, V):
    assert L % 8 == 0 and B == 1024 and V % _BLK == 0
    tile_rows = L // 8
    b_per_w = B // _NUM_WORKERS
    groups = b_per_w // _LANES
    n_blocks = V // _BLK
    blk_per_tile = 200
    f32_per_tile = blk_per_tile * _BLK
    words_per_tile = blk_per_tile * _LANES
    stage_w = _NUM_SUBCORES * words_per_tile
    mesh = plsc.VectorSubcoreMesh(core_axis_name="c", subcore_axis_name="s")

    @functools.partial(
        pl.kernel,
        mesh=mesh,
        out_type=(
            jax.ShapeDtypeStruct((1, B), jnp.float32),
            jax.ShapeDtypeStruct((_NUM_CORES, stage_w), jnp.int32),
        ),
        scratch_types=[
            pltpu.VMEM((V // 2,), jnp.int32),
            pltpu.VMEM((tile_rows, 8, b_per_w), jnp.int32),
            pltpu.VMEM((f32_per_tile,), jnp.float32),
            pltpu.VMEM((words_per_tile,), jnp.int32),
            pltpu.VMEM((b_per_w,), jnp.float32),
            pltpu.VMEM((_LANES,), jnp.float32),
            pltpu.SemaphoreType.DMA,
            pltpu.SemaphoreType.DMA,
        ],
        compiler_params=pltpu.CompilerParams(
            needs_layout_passes=False, use_tc_tiling_on_sc=False),
    )
    def gather_sum(w_hbm, tex_hbm, bias_hbm, out_hbm, stage_hbm, w_v, tex_v,
                   pin_v, pout_v, out_v, bias_v, sem_w, sem_t):
        core = lax.axis_index("c")
        sid = lax.axis_index("s")
        wid = sid * _NUM_CORES + core
        tile_col = wid // 4
        c0 = (wid % 4) * b_per_w
        cp_t = pltpu.async_copy(
            tex_hbm.at[:, tile_col, :, pl.ds(c0, b_per_w)], tex_v, sem_t)
        pltpu.sync_copy(bias_hbm, bias_v.at[pl.ds(0, 1)])

        blk0 = sid * blk_per_tile
        f32_start = blk0 * _BLK
        dma_start = jnp.minimum(f32_start, V - f32_per_tile)
        local_off = f32_start - dma_start
        pltpu.sync_copy(w_hbm.at[0, pl.ds(dma_start, f32_per_tile)], pin_v)

        def pack_body(i, carry):
            @pl.when(blk0 + i < n_blocks)
            def _():
                base = local_off + i * _BLK
                b0 = plsc.bitcast(pin_v[pl.ds(base, _LANES)], jnp.int32)
                b1 = plsc.bitcast(pin_v[pl.ds(base + _LANES, _LANES)],
                                  jnp.int32)
                pout_v[pl.ds(i * _LANES, _LANES)] = (
                    _rne16(b0) | (_rne16(b1) << 16))
            return carry

        lax.fori_loop(0, blk_per_tile, pack_body, 0)
        pltpu.sync_copy(pout_v,
                        stage_hbm.at[core, pl.ds(sid * words_per_tile,
                                                 words_per_tile)])
        plsc.subcore_barrier()

        cp_w = pltpu.async_copy(stage_hbm.at[core, pl.ds(0, V // 2)], w_v,
                                sem_w)
        cp_t.wait()
        cp_w.wait()
        bias = bias_v[...][0]

        def body(tr, accs):
            out = list(accs)
            for sub in range(8):
                for g in range(groups):
                    idx = tex_v[tr, sub, pl.ds(g * _LANES, _LANES)]
                    gidx = ((idx >> 5) << 4) | (idx & 15)
                    word = plsc.load_gather(w_v, [gidx])
                    bits = jnp.where((idx & 16) == 16, word & _HI_MASK,
                                     word << 16)
                    out[g] = out[g] + plsc.bitcast(bits, jnp.float32)
            return tuple(out)

        init = tuple(jnp.zeros((_LANES,), jnp.float32) for _ in range(groups))
        accs = lax.fori_loop(0, tile_rows, body, init)
        for g in range(groups):
            out_v[pl.ds(g * _LANES, _LANES)] = accs[g] + bias
        pltpu.sync_copy(
            out_v, out_hbm.at[0, pl.ds(tile_col * 128 + c0, b_per_w)])

    return gather_sum


def kernel(text, W, b):
    L, B = text.shape
    V = W.shape[1]
    tex4 = text.reshape(L // 8, 8, B // 128, 128).transpose(0, 2, 1, 3)
    out, _ = _make_gather_sum(L, B, V)(W, tex4, b)
    return out.reshape(B, 1)
